# Initial kernel scaffold; baseline (speedup 1.0000x reference)
import jax
import jax.numpy as jnp
from jax.experimental import pallas as pl


def kernel(rpn_cls_score, gt_boxes, im_info):
    raise NotImplementedError("write your pallas kernel here")


# speedup vs baseline: 2.2965x; 2.2965x over previous
import numpy as np
import jax
import jax.numpy as jnp
from jax.experimental import pallas as pl
from jax.experimental.pallas import tpu as pltpu

_FEAT_STRIDE = 16
_NEG_OV = 0.3
_POS_OV = 0.7
_BATCHSIZE = 256
_NUM_FG = 128
_A = 9
_H = 128
_W = 128
_G = 20
_K = _H * _W
_N = _K * _A
_ROWS = _N // 128
_RT = 128
_T = _ROWS // _RT


def _gen_base_anchors():
    base = np.array([0.0, 0.0, 15.0, 15.0])
    w = base[2] - base[0] + 1
    h = base[3] - base[1] + 1
    xc = base[0] + 0.5 * (w - 1)
    yc = base[1] + 0.5 * (h - 1)
    size = w * h
    out = []
    for r in (0.5, 1.0, 2.0):
        ws0 = np.round(np.sqrt(size / r))
        hs0 = np.round(ws0 * r)
        for s in (8.0, 16.0, 32.0):
            ws = ws0 * s
            hs = hs0 * s
            out.append([xc - 0.5 * (ws - 1), yc - 0.5 * (hs - 1),
                        xc + 0.5 * (ws - 1), yc + 0.5 * (hs - 1)])
    return np.array(out, dtype=np.float64)


_BASE = _gen_base_anchors()
_sx = np.arange(_W) * _FEAT_STRIDE
_sy = np.arange(_H) * _FEAT_STRIDE
_sx, _sy = np.meshgrid(_sx, _sy)
_SHIFTS = np.vstack((_sx.ravel(), _sy.ravel(), _sx.ravel(), _sy.ravel())).T
_ALL = (_BASE.astype(np.float32).reshape(1, _A, 4)
        + _SHIFTS.astype(np.float32).reshape(_K, 1, 4)).reshape(_N, 4)
_IMS = _H * _FEAT_STRIDE
_INSIDE_NP = ((_ALL[:, 0] >= 0) & (_ALL[:, 1] >= 0)
              & (_ALL[:, 2] < _IMS) & (_ALL[:, 3] < _IMS))
_NIN = int(_INSIDE_NP.sum())
_STREAM_NP = np.random.RandomState(42).randint(
    0, 2 ** 32, size=12 * 4 * (_NIN + 2), dtype=np.uint32)

_AX1 = _ALL[:, 0].reshape(_ROWS, 128)
_AY1 = _ALL[:, 1].reshape(_ROWS, 128)
_AX2 = _ALL[:, 2].reshape(_ROWS, 128)
_AY2 = _ALL[:, 3].reshape(_ROWS, 128)
_INS = _INSIDE_NP.astype(np.float32).reshape(_ROWS, 128)


def _iou_tile(x1, y1, x2, y2, gx1, gy1, gx2, gy2):
    aarea = (x2 - x1 + 1.0) * (y2 - y1 + 1.0)
    garea = (gx2 - gx1 + 1.0) * (gy2 - gy1 + 1.0)
    iw = jnp.minimum(x2, gx2) - jnp.maximum(x1, gx1) + 1.0
    ih = jnp.minimum(y2, gy2) - jnp.maximum(y1, gy1) + 1.0
    iw = jnp.maximum(iw, 0.0)
    ih = jnp.maximum(ih, 0.0)
    ua = aarea + garea - iw * ih
    return (iw * ih) / ua


def _kernel_a(x1r, y1r, x2r, y2r, insr, gtr, maxov_r, argm_r, gtmax_r):
    t = pl.program_id(1)
    x1 = x1r[...]
    y1 = y1r[...]
    x2 = x2r[...]
    y2 = y2r[...]
    ins = insr[...]
    lane = jax.lax.broadcasted_iota(jnp.int32, (8, 128), 1)
    cur_max = jnp.full((_RT, 128), -1.0, dtype=jnp.float32)
    cur_arg = jnp.zeros((_RT, 128), dtype=jnp.int32)
    acc = jnp.where(t == 0, jnp.zeros((8, 128), jnp.float32), gtmax_r[0, :, :])
    for g in range(_G):
        gx1 = gtr[0, g, 0]
        gy1 = gtr[0, g, 1]
        gx2 = gtr[0, g, 2]
        gy2 = gtr[0, g, 3]
        iou = _iou_tile(x1, y1, x2, y2, gx1, gy1, gx2, gy2)
        upd = iou > cur_max
        cur_arg = jnp.where(upd, g, cur_arg)
        cur_max = jnp.where(upd, iou, cur_max)
        mg = jnp.max(iou * ins, axis=0, keepdims=True)
        mg = jnp.max(mg, axis=1, keepdims=True)
        acc = jnp.where(lane == g, jnp.maximum(acc, mg), acc)
    maxov_r[0, :, :] = cur_max
    argm_r[0, :, :] = cur_arg
    gtmax_r[0, :, :] = acc


def _kernel_b(x1r, y1r, x2r, y2r, insr, gtr, gtmaxr, maxovr, argmr,
              lab_r, dx_r, dy_r, dw_r, dh_r, cnt_r):
    t = pl.program_id(1)
    x1 = x1r[...]
    y1 = y1r[...]
    x2 = x2r[...]
    y2 = y2r[...]
    ins = insr[...]
    insb = ins > 0.0
    mo = maxovr[0, :, :]
    am = argmr[0, :, :]
    lane = jax.lax.broadcasted_iota(jnp.int32, (8, 128), 1)
    row = jax.lax.broadcasted_iota(jnp.int32, (8, 128), 0)

    label = jnp.where(mo < _NEG_OV, 0.0, -1.0)
    keep = jnp.zeros((_RT, 128), dtype=jnp.bool_)
    gsx1 = jnp.zeros((_RT, 128), dtype=jnp.float32)
    gsy1 = jnp.zeros((_RT, 128), dtype=jnp.float32)
    gsx2 = jnp.zeros((_RT, 128), dtype=jnp.float32)
    gsy2 = jnp.zeros((_RT, 128), dtype=jnp.float32)
    for g in range(_G):
        gx1 = gtr[0, g, 0]
        gy1 = gtr[0, g, 1]
        gx2 = gtr[0, g, 2]
        gy2 = gtr[0, g, 3]
        iou = _iou_tile(x1, y1, x2, y2, gx1, gy1, gx2, gy2)
        gm = gtmaxr[0, 0, g]
        gm = jnp.where(gm == 0.0, 1e-5, gm)
        keep = keep | (iou == gm)
        sel = am == g
        gsx1 = jnp.where(sel, gx1, gsx1)
        gsy1 = jnp.where(sel, gy1, gsy1)
        gsx2 = jnp.where(sel, gx2, gsx2)
        gsy2 = jnp.where(sel, gy2, gsy2)
    label = jnp.where(keep, 1.0, label)
    label = jnp.where(mo >= _POS_OV, 1.0, label)
    label = jnp.where(insb, label, -1.0)
    lab_r[0, :, :] = label

    cf = jnp.sum((label == 1.0).astype(jnp.float32), axis=0, keepdims=True)
    cf = jnp.sum(cf, axis=1, keepdims=True)
    cb = jnp.sum((label == 0.0).astype(jnp.float32), axis=0, keepdims=True)
    cb = jnp.sum(cb, axis=1, keepdims=True)
    prev = jnp.where(t == 0, jnp.zeros((8, 128), jnp.float32), cnt_r[0, :, :])
    zero = jnp.zeros((8, 128), jnp.float32)
    sel0 = (lane == 0) & (row == 0)
    sel1 = (lane == 1) & (row == 0)
    cnt_r[0, :, :] = prev + jnp.where(sel0, cf, zero) + jnp.where(sel1, cb, zero)

    ex_w = x2 - x1 + 1.0
    ex_h = y2 - y1 + 1.0
    ex_cx = x1 + 0.5 * ex_w
    ex_cy = y1 + 0.5 * ex_h
    gt_w = gsx2 - gsx1 + 1.0
    gt_h = gsy2 - gsy1 + 1.0
    gt_cx = gsx1 + 0.5 * gt_w
    gt_cy = gsy1 + 0.5 * gt_h
    dx_r[0, :, :] = ((gt_cx - ex_cx) / ex_w) * ins
    dy_r[0, :, :] = ((gt_cy - ex_cy) / ex_h) * ins
    dw_r[0, :, :] = jnp.log(gt_w / ex_w) * ins
    dh_r[0, :, :] = jnp.log(gt_h / ex_h) * ins


def _kernel_c(labr, poswr, biw_r, bow_r):
    lab = labr[0, :, :]
    pw = poswr[0, 0, 0]
    biw_r[0, :, :] = jnp.where(lab == 1.0, 1.0, 0.0)
    bow_r[0, :, :] = jnp.where(lab >= 0.0, pw, 0.0)


def _smear(u):
    u = u | (u >> 1)
    u = u | (u >> 2)
    u = u | (u >> 4)
    u = u | (u >> 8)
    u = u | (u >> 16)
    return u


def _shuffle(n, cursor, stream, n_static):
    perm0 = jnp.arange(n_static, dtype=jnp.int32)

    def cond(c):
        return c[2] >= 1

    def body(c):
        perm, cur, i = c
        i_u = i.astype(jnp.uint32)
        m = _smear(i_u)
        v = stream[cur] & m
        cur = cur + 1
        acc = v <= i_u
        j = v.astype(jnp.int32)
        pi = perm[i]
        pj = perm[j]
        idx_i = jnp.where(acc, i, n_static)
        idx_j = jnp.where(acc, j, n_static)
        perm = perm.at[idx_i].set(pj, mode='drop').at[idx_j].set(pi, mode='drop')
        i = jnp.where(acc, i - 1, i)
        return (perm, cur, i)

    perm, cur, _ = jax.lax.while_loop(
        cond, body, (perm0, cursor, n - 1))
    return perm, cur


def kernel(rpn_cls_score, gt_boxes, im_info):
    B = gt_boxes.shape[0]
    x1 = jnp.asarray(_AX1)
    y1 = jnp.asarray(_AY1)
    x2 = jnp.asarray(_AX2)
    y2 = jnp.asarray(_AY2)
    ins = jnp.asarray(_INS)
    stream = jnp.asarray(_STREAM_NP)

    anch_spec = pl.BlockSpec((_RT, 128), lambda b, t: (t, 0))
    gt_spec = pl.BlockSpec((1, _G, 5), lambda b, t: (b, 0, 0),
                           memory_space=pltpu.SMEM)
    tile_out = pl.BlockSpec((1, _RT, 128), lambda b, t: (b, t, 0))
    bvec_out = pl.BlockSpec((1, 8, 128), lambda b, t: (b, 0, 0))

    maxov, argm, gtmax = pl.pallas_call(
        _kernel_a,
        grid=(B, _T),
        in_specs=[anch_spec] * 5 + [gt_spec],
        out_specs=[tile_out, tile_out, bvec_out],
        out_shape=[
            jax.ShapeDtypeStruct((B, _ROWS, 128), jnp.float32),
            jax.ShapeDtypeStruct((B, _ROWS, 128), jnp.int32),
            jax.ShapeDtypeStruct((B, 8, 128), jnp.float32),
        ],
    )(x1, y1, x2, y2, ins, gt_boxes)

    gtmax_spec = pl.BlockSpec((1, 8, 128), lambda b, t: (b, 0, 0),
                              memory_space=pltpu.SMEM)
    lab0, dx, dy, dw, dh, cnts = pl.pallas_call(
        _kernel_b,
        grid=(B, _T),
        in_specs=[anch_spec] * 5 + [gt_spec, gtmax_spec, tile_out, tile_out],
        out_specs=[tile_out] * 5 + [bvec_out],
        out_shape=[jax.ShapeDtypeStruct((B, _ROWS, 128), jnp.float32)] * 5
        + [jax.ShapeDtypeStruct((B, 8, 128), jnp.float32)],
    )(x1, y1, x2, y2, ins, gt_boxes, gtmax, maxov, argm)

    lab0 = lab0.reshape(B, _N)
    pos_nin = jnp.arange(_NIN, dtype=jnp.int32)
    posN = jnp.arange(_N, dtype=jnp.int32)
    cursor = jnp.int32(0)
    labs = []
    posws = []
    for i in range(B):
        lab = lab0[i]
        sum_fg = cnts[i, 0, 0].astype(jnp.int32)
        sum_bg = cnts[i, 0, 1].astype(jnp.int32)
        do_fg = sum_fg > _NUM_FG
        fg_order = jnp.argsort(jnp.where(lab == 1.0, posN, posN + _N))
        perm_fg, cursor = _shuffle(jnp.where(do_fg, sum_fg, 0), cursor,
                                   stream, _NIN)
        cnt_fg = jnp.where(do_fg, sum_fg - _NUM_FG, 0)
        fg_targets = jnp.where(pos_nin < cnt_fg, fg_order[perm_fg], _N)
        lab = lab.at[fg_targets].set(-1.0, mode='drop')
        sum_fg_kept = jnp.minimum(sum_fg, _NUM_FG)
        num_bg = _BATCHSIZE - sum_fg_kept
        do_bg = sum_bg > num_bg
        bg_order = jnp.argsort(jnp.where(lab == 0.0, posN, posN + _N))
        perm_bg, cursor = _shuffle(jnp.where(do_bg, sum_bg, 0), cursor,
                                   stream, _NIN)
        cnt_bg = jnp.where(do_bg, sum_bg - num_bg, 0)
        bg_targets = jnp.where(pos_nin < cnt_bg, bg_order[perm_bg], _N)
        lab = lab.at[bg_targets].set(-1.0, mode='drop')
        bg_kept = jnp.minimum(sum_bg, num_bg)
        num_ex = (sum_fg_kept + bg_kept).astype(jnp.float32)
        labs.append(lab)
        posws.append(1.0 / num_ex)
    labels = jnp.stack(labs, axis=0)
    posw = jnp.stack(posws, axis=0).reshape(B, 1, 1)

    lab3 = labels.reshape(B, _ROWS, 128)
    posw_spec = pl.BlockSpec((1, 1, 1), lambda b, t: (b, 0, 0),
                             memory_space=pltpu.SMEM)
    biw, bow = pl.pallas_call(
        _kernel_c,
        grid=(B, _T),
        in_specs=[tile_out, posw_spec],
        out_specs=[tile_out, tile_out],
        out_shape=[jax.ShapeDtypeStruct((B, _ROWS, 128), jnp.float32)] * 2,
    )(lab3, posw)

    labels_out = labels.reshape(B, _H, _W, _A).transpose(0, 3, 1, 2)
    labels_out = labels_out.reshape(B, 1, _A * _H, _W)
    tgt = jnp.stack([dx.reshape(B, _N), dy.reshape(B, _N),
                     dw.reshape(B, _N), dh.reshape(B, _N)], axis=2)
    tgt_out = tgt.reshape(B, _H, _W, _A * 4).transpose(0, 3, 1, 2)
    biw_out = jnp.broadcast_to(biw.reshape(B, _N, 1), (B, _N, 4))
    biw_out = biw_out.reshape(B, _H, _W, 4 * _A).transpose(0, 3, 1, 2)
    bow_out = jnp.broadcast_to(bow.reshape(B, _N, 1), (B, _N, 4))
    bow_out = bow_out.reshape(B, _H, _W, 4 * _A).transpose(0, 3, 1, 2)
    return (labels_out, tgt_out, biw_out, bow_out)
